# baseline (device time: 455394 ns/iter reference)
import jax
import jax.numpy as jnp
from jax import lax
from jax.experimental import pallas as pl
from jax.experimental.pallas import tpu as pltpu

N_DEV = 4
SQ = 256
HL = 8
DH = 128
DM = 1024
SKV_L = 4096
SCALE = 0.08838834764831843

_CompilerParams = getattr(pltpu, "CompilerParams", None) or getattr(
    pltpu, "TPUCompilerParams"
)


def _neighbor_barrier(me):
    bar = pltpu.get_barrier_semaphore()
    for k in range(1, N_DEV):
        pl.semaphore_signal(
            bar,
            inc=1,
            device_id=((me + k) % N_DEV,),
            device_id_type=pl.DeviceIdType.MESH,
        )
    pl.semaphore_wait(bar, N_DEV - 1)


def _qproj_allgather(x, Wq):

    def body(x_ref, wq_ref, out_ref, send_sems, recv_sems):
        me = lax.axis_index("i")
        _neighbor_barrier(me)
        q = jnp.dot(x_ref[0], wq_ref[...], preferred_element_type=jnp.float32)
        out_ref[me] = q
        rdmas = []
        for k in range(1, N_DEV):
            r = pltpu.make_async_remote_copy(
                src_ref=out_ref.at[me],
                dst_ref=out_ref.at[me],
                send_sem=send_sems.at[k - 1],
                recv_sem=recv_sems.at[k - 1],
                device_id=((me + k) % N_DEV,),
                device_id_type=pl.DeviceIdType.MESH,
            )
            r.start()
            rdmas.append(r)
        for r in rdmas:
            r.wait_send()
        for r in rdmas:
            r.wait_recv()

    return pl.pallas_call(
        body,
        out_shape=jax.ShapeDtypeStruct((N_DEV, SQ, DM), jnp.float32),
        in_specs=[
            pl.BlockSpec(memory_space=pltpu.VMEM),
            pl.BlockSpec(memory_space=pltpu.VMEM),
        ],
        out_specs=pl.BlockSpec(memory_space=pltpu.VMEM),
        scratch_shapes=[
            pltpu.SemaphoreType.DMA((N_DEV - 1,)),
            pltpu.SemaphoreType.DMA((N_DEV - 1,)),
        ],
        compiler_params=_CompilerParams(collective_id=0),
    )(x, Wq)


def _attn_partials(Qg, K_ext, V_ext):

    SKV_C = 1024
    NT = SKV_L // SKV_C

    def body(q_ref, k_ref, v_ref, acc_ref, ml_ref):
        t = pl.program_id(1)
        for h in range(HL):
            k4 = k_ref[0, :, h, :].reshape(4, 4, 64, DH)
            v4 = v_ref[0, :, h, :].reshape(4, 4, 64, DH)
            for qb in range(4):
                q = q_ref[0][qb * 64:(qb + 1) * 64, h * DH:(h + 1) * DH]
                k_sel = k4[:, qb].reshape(4 * 64, DH)
                v_sel = v4[:, qb].reshape(4 * 64, DH)
                s = (
                    lax.dot_general(
                        q, k_sel, (((1,), (1,)), ((), ())),
                        preferred_element_type=jnp.float32,
                    )
                    * SCALE
                )
                m_t = jnp.max(s, axis=1)
                w = jnp.exp(s - m_t[:, None])
                l_t = jnp.sum(w, axis=1)
                pv = lax.dot_general(
                    w, v_sel, (((1,), (0,)), ((), ())),
                    preferred_element_type=jnp.float32,
                )
                r0, r1 = qb * 64, (qb + 1) * 64

                @pl.when(t == 0)
                def _(h=h, r0=r0, r1=r1, m_t=m_t, l_t=l_t, pv=pv):
                    acc_ref[0, h, r0:r1, :] = pv
                    ml_ref[0, h, 0, r0:r1] = m_t
                    ml_ref[0, h, 1, r0:r1] = l_t

                @pl.when(t > 0)
                def _(h=h, r0=r0, r1=r1, m_t=m_t, l_t=l_t, pv=pv):
                    m_old = ml_ref[0, h, 0, r0:r1]
                    l_old = ml_ref[0, h, 1, r0:r1]
                    m_new = jnp.maximum(m_old, m_t)
                    a_old = jnp.exp(m_old - m_new)
                    a_t = jnp.exp(m_t - m_new)
                    acc_ref[0, h, r0:r1, :] = (
                        acc_ref[0, h, r0:r1, :] * a_old[:, None]
                        + pv * a_t[:, None]
                    )
                    ml_ref[0, h, 0, r0:r1] = m_new
                    ml_ref[0, h, 1, r0:r1] = l_old * a_old + l_t * a_t

    return pl.pallas_call(
        body,
        grid=(N_DEV, NT),
        in_specs=[
            pl.BlockSpec((1, SQ, DM), lambda j, t: (j, 0, 0)),
            pl.BlockSpec((1, SKV_C, HL, DH), lambda j, t: (0, t, j, 0)),
            pl.BlockSpec((1, SKV_C, HL, DH), lambda j, t: (0, t, j, 0)),
        ],
        out_specs=[
            pl.BlockSpec((1, HL, SQ, DH), lambda j, t: (j, 0, 0, 0)),
            pl.BlockSpec((1, HL, 2, SQ), lambda j, t: (j, 0, 0, 0)),
        ],
        out_shape=[
            jax.ShapeDtypeStruct((N_DEV, HL, SQ, DH), jnp.float32),
            jax.ShapeDtypeStruct((N_DEV, HL, 2, SQ), jnp.float32),
        ],
    )(Qg, K_ext, V_ext)


def _combine_wo_allreduce(acc, ml, Wo):

    def body(
        acc_ref, ml_ref, wo_ref, out_ref,
        acc_rx, ml_rx, out_rx,
        a_send, a_recv, m_send, m_recv, o_send, o_recv,
    ):
        me = lax.axis_index("i")
        _neighbor_barrier(me)

        acc_rx[me] = acc_ref[me]
        ml_rx[me] = ml_ref[me]
        rdmas = []
        for k in range(1, N_DEV):
            p = (me + k) % N_DEV
            r1 = pltpu.make_async_remote_copy(
                src_ref=acc_ref.at[p],
                dst_ref=acc_rx.at[me],
                send_sem=a_send.at[k - 1],
                recv_sem=a_recv.at[k - 1],
                device_id=(p,),
                device_id_type=pl.DeviceIdType.MESH,
            )
            r2 = pltpu.make_async_remote_copy(
                src_ref=ml_ref.at[p],
                dst_ref=ml_rx.at[me],
                send_sem=m_send.at[k - 1],
                recv_sem=m_recv.at[k - 1],
                device_id=(p,),
                device_id_type=pl.DeviceIdType.MESH,
            )
            r1.start()
            r2.start()
            rdmas += [r1, r2]
        for r in rdmas:
            r.wait_send()
        for r in rdmas:
            r.wait_recv()

        m_all = ml_rx[:, :, 0, :]
        big_m = jnp.max(m_all, axis=0)
        scale = jnp.exp(m_all - big_m[None])
        l = jnp.sum(ml_rx[:, :, 1, :] * scale, axis=0)
        ctx = jnp.sum(acc_rx[...] * scale[..., None], axis=0)
        ctx = ctx / l[..., None]

        o = jnp.zeros((SQ, DM), jnp.float32)
        for h in range(HL):
            o = o + jnp.dot(
                ctx[h],
                wo_ref[h * DH:(h + 1) * DH, :],
                preferred_element_type=jnp.float32,
            )

        out_rx[me] = o
        rdmas2 = []
        for k in range(1, N_DEV):
            p = (me + k) % N_DEV
            r = pltpu.make_async_remote_copy(
                src_ref=out_rx.at[me],
                dst_ref=out_rx.at[me],
                send_sem=o_send.at[k - 1],
                recv_sem=o_recv.at[k - 1],
                device_id=(p,),
                device_id_type=pl.DeviceIdType.MESH,
            )
            r.start()
            rdmas2.append(r)
        for r in rdmas2:
            r.wait_send()
        for r in rdmas2:
            r.wait_recv()
        out_ref[...] = jnp.sum(out_rx[...], axis=0)

    return pl.pallas_call(
        body,
        out_shape=jax.ShapeDtypeStruct((SQ, DM), jnp.float32),
        in_specs=[
            pl.BlockSpec(memory_space=pltpu.VMEM),
            pl.BlockSpec(memory_space=pltpu.VMEM),
            pl.BlockSpec(memory_space=pltpu.VMEM),
        ],
        out_specs=pl.BlockSpec(memory_space=pltpu.VMEM),
        scratch_shapes=[
            pltpu.VMEM((N_DEV, HL, SQ, DH), jnp.float32),
            pltpu.VMEM((N_DEV, HL, 2, SQ), jnp.float32),
            pltpu.VMEM((N_DEV, SQ, DM), jnp.float32),
            pltpu.SemaphoreType.DMA((N_DEV - 1,)),
            pltpu.SemaphoreType.DMA((N_DEV - 1,)),
            pltpu.SemaphoreType.DMA((N_DEV - 1,)),
            pltpu.SemaphoreType.DMA((N_DEV - 1,)),
            pltpu.SemaphoreType.DMA((N_DEV - 1,)),
            pltpu.SemaphoreType.DMA((N_DEV - 1,)),
        ],
        compiler_params=_CompilerParams(collective_id=1),
    )(acc, ml, Wo)


def kernel(x, Wq, K_ext, V_ext, Wo):
    Qg = _qproj_allgather(x, Wq)
    acc, ml = _attn_partials(Qg, K_ext, V_ext)
    out = _combine_wo_allreduce(acc, ml, Wo)
    return out.reshape(1, SQ, DM)


# device time: 375176 ns/iter; 1.2138x vs baseline; 1.2138x over previous
import jax
import jax.numpy as jnp
from jax import lax
from jax.experimental import pallas as pl
from jax.experimental.pallas import tpu as pltpu

N_DEV = 4
SQ = 256
HL = 8
DH = 128
DM = 1024
SKV_L = 4096
SCALE = 0.08838834764831843

_CompilerParams = getattr(pltpu, "CompilerParams", None) or getattr(
    pltpu, "TPUCompilerParams"
)


def _neighbor_barrier(me):
    bar = pltpu.get_barrier_semaphore()
    for k in range(1, N_DEV):
        pl.semaphore_signal(
            bar,
            inc=1,
            device_id=((me + k) % N_DEV,),
            device_id_type=pl.DeviceIdType.MESH,
        )
    pl.semaphore_wait(bar, N_DEV - 1)


def _qproj_allgather(x, Wq):

    def body(x_ref, wq_ref, out_ref, send_sems, recv_sems):
        me = lax.axis_index("i")
        _neighbor_barrier(me)
        q = jnp.dot(
            x_ref[0].astype(jnp.bfloat16),
            wq_ref[...].astype(jnp.bfloat16),
            preferred_element_type=jnp.float32,
        )
        out_ref[me] = q
        rdmas = []
        for k in range(1, N_DEV):
            r = pltpu.make_async_remote_copy(
                src_ref=out_ref.at[me],
                dst_ref=out_ref.at[me],
                send_sem=send_sems.at[k - 1],
                recv_sem=recv_sems.at[k - 1],
                device_id=((me + k) % N_DEV,),
                device_id_type=pl.DeviceIdType.MESH,
            )
            r.start()
            rdmas.append(r)
        for r in rdmas:
            r.wait_send()
        for r in rdmas:
            r.wait_recv()

    return pl.pallas_call(
        body,
        out_shape=jax.ShapeDtypeStruct((N_DEV, SQ, DM), jnp.float32),
        in_specs=[
            pl.BlockSpec(memory_space=pltpu.VMEM),
            pl.BlockSpec(memory_space=pltpu.VMEM),
        ],
        out_specs=pl.BlockSpec(memory_space=pltpu.VMEM),
        scratch_shapes=[
            pltpu.SemaphoreType.DMA((N_DEV - 1,)),
            pltpu.SemaphoreType.DMA((N_DEV - 1,)),
        ],
        compiler_params=_CompilerParams(collective_id=0),
    )(x, Wq)


def _attn_partials(Qg, K_ext, V_ext):

    SKV_C = 1024
    NT = SKV_L // SKV_C

    def body(q_ref, k_ref, v_ref, acc_ref, ml_ref):
        t = pl.program_id(1)
        rows = lax.broadcasted_iota(jnp.int32, (SQ, SKV_C), 0)
        cols = lax.broadcasted_iota(jnp.int32, (SQ, SKV_C), 1)
        mask = ((cols // 64) % 4) == (rows // 64)
        for h in range(HL):
            q = q_ref[0][:, h * DH:(h + 1) * DH].astype(jnp.bfloat16)
            k = k_ref[0, :, h, :].astype(jnp.bfloat16)
            v = v_ref[0, :, h, :].astype(jnp.bfloat16)
            s = (
                lax.dot_general(
                    q, k, (((1,), (1,)), ((), ())),
                    preferred_element_type=jnp.float32,
                )
                * SCALE
            )
            s = jnp.where(mask, s, -1e9)
            m_t = jnp.max(s, axis=1)
            w = jnp.exp(s - m_t[:, None])
            l_t = jnp.sum(w, axis=1)
            pv = lax.dot_general(
                w.astype(jnp.bfloat16), v, (((1,), (0,)), ((), ())),
                preferred_element_type=jnp.float32,
            )

            @pl.when(t == 0)
            def _(h=h, m_t=m_t, l_t=l_t, pv=pv):
                acc_ref[0, h] = pv
                ml_ref[0, h, 0] = m_t
                ml_ref[0, h, 1] = l_t

            @pl.when(t > 0)
            def _(h=h, m_t=m_t, l_t=l_t, pv=pv):
                m_old = ml_ref[0, h, 0]
                l_old = ml_ref[0, h, 1]
                m_new = jnp.maximum(m_old, m_t)
                a_old = jnp.exp(m_old - m_new)
                a_t = jnp.exp(m_t - m_new)
                acc_ref[0, h] = (
                    acc_ref[0, h] * a_old[:, None] + pv * a_t[:, None]
                )
                ml_ref[0, h, 0] = m_new
                ml_ref[0, h, 1] = l_old * a_old + l_t * a_t

    return pl.pallas_call(
        body,
        grid=(N_DEV, NT),
        in_specs=[
            pl.BlockSpec((1, SQ, DM), lambda j, t: (j, 0, 0)),
            pl.BlockSpec((1, SKV_C, HL, DH), lambda j, t: (0, t, j, 0)),
            pl.BlockSpec((1, SKV_C, HL, DH), lambda j, t: (0, t, j, 0)),
        ],
        out_specs=[
            pl.BlockSpec((1, HL, SQ, DH), lambda j, t: (j, 0, 0, 0)),
            pl.BlockSpec((1, HL, 2, SQ), lambda j, t: (j, 0, 0, 0)),
        ],
        out_shape=[
            jax.ShapeDtypeStruct((N_DEV, HL, SQ, DH), jnp.float32),
            jax.ShapeDtypeStruct((N_DEV, HL, 2, SQ), jnp.float32),
        ],
    )(Qg, K_ext, V_ext)


def _combine_wo_allreduce(acc, ml, Wo):

    def body(
        acc_ref, ml_ref, wo_ref, out_ref,
        acc_rx, ml_rx, out_rx,
        a_send, a_recv, m_send, m_recv, o_send, o_recv,
    ):
        me = lax.axis_index("i")
        _neighbor_barrier(me)

        acc_rx[me] = acc_ref[me]
        ml_rx[me] = ml_ref[me]
        rdmas = []
        for k in range(1, N_DEV):
            p = (me + k) % N_DEV
            r1 = pltpu.make_async_remote_copy(
                src_ref=acc_ref.at[p],
                dst_ref=acc_rx.at[me],
                send_sem=a_send.at[k - 1],
                recv_sem=a_recv.at[k - 1],
                device_id=(p,),
                device_id_type=pl.DeviceIdType.MESH,
            )
            r2 = pltpu.make_async_remote_copy(
                src_ref=ml_ref.at[p],
                dst_ref=ml_rx.at[me],
                send_sem=m_send.at[k - 1],
                recv_sem=m_recv.at[k - 1],
                device_id=(p,),
                device_id_type=pl.DeviceIdType.MESH,
            )
            r1.start()
            r2.start()
            rdmas += [r1, r2]
        for r in rdmas:
            r.wait_send()
        for r in rdmas:
            r.wait_recv()

        m_all = ml_rx[:, :, 0, :]
        big_m = jnp.max(m_all, axis=0)
        scale = jnp.exp(m_all - big_m[None])
        l = jnp.sum(ml_rx[:, :, 1, :] * scale, axis=0)
        ctx = jnp.sum(acc_rx[...] * scale[..., None], axis=0)
        ctx = ctx / l[..., None]

        o = jnp.zeros((SQ, DM), jnp.float32)
        for h in range(HL):
            o = o + jnp.dot(
                ctx[h].astype(jnp.bfloat16),
                wo_ref[h * DH:(h + 1) * DH, :].astype(jnp.bfloat16),
                preferred_element_type=jnp.float32,
            )

        out_rx[me] = o
        rdmas2 = []
        for k in range(1, N_DEV):
            p = (me + k) % N_DEV
            r = pltpu.make_async_remote_copy(
                src_ref=out_rx.at[me],
                dst_ref=out_rx.at[me],
                send_sem=o_send.at[k - 1],
                recv_sem=o_recv.at[k - 1],
                device_id=(p,),
                device_id_type=pl.DeviceIdType.MESH,
            )
            r.start()
            rdmas2.append(r)
        for r in rdmas2:
            r.wait_send()
        for r in rdmas2:
            r.wait_recv()
        out_ref[...] = jnp.sum(out_rx[...], axis=0)

    return pl.pallas_call(
        body,
        out_shape=jax.ShapeDtypeStruct((SQ, DM), jnp.float32),
        in_specs=[
            pl.BlockSpec(memory_space=pltpu.VMEM),
            pl.BlockSpec(memory_space=pltpu.VMEM),
            pl.BlockSpec(memory_space=pltpu.VMEM),
        ],
        out_specs=pl.BlockSpec(memory_space=pltpu.VMEM),
        scratch_shapes=[
            pltpu.VMEM((N_DEV, HL, SQ, DH), jnp.float32),
            pltpu.VMEM((N_DEV, HL, 2, SQ), jnp.float32),
            pltpu.VMEM((N_DEV, SQ, DM), jnp.float32),
            pltpu.SemaphoreType.DMA((N_DEV - 1,)),
            pltpu.SemaphoreType.DMA((N_DEV - 1,)),
            pltpu.SemaphoreType.DMA((N_DEV - 1,)),
            pltpu.SemaphoreType.DMA((N_DEV - 1,)),
            pltpu.SemaphoreType.DMA((N_DEV - 1,)),
            pltpu.SemaphoreType.DMA((N_DEV - 1,)),
        ],
        compiler_params=_CompilerParams(collective_id=1),
    )(acc, ml, Wo)


def kernel(x, Wq, K_ext, V_ext, Wo):
    Qg = _qproj_allgather(x, Wq)
    acc, ml = _attn_partials(Qg, K_ext, V_ext)
    out = _combine_wo_allreduce(acc, ml, Wo)
    return out.reshape(1, SQ, DM)


# device time: 178606 ns/iter; 2.5497x vs baseline; 2.1006x over previous
import jax
import jax.numpy as jnp
from jax import lax
from jax.experimental import pallas as pl
from jax.experimental.pallas import tpu as pltpu

N_DEV = 4
SQ = 256
HL = 8
DH = 128
DM = 1024
SKV_L = 4096
SCALE = 0.08838834764831843

_CompilerParams = getattr(pltpu, "CompilerParams", None) or getattr(
    pltpu, "TPUCompilerParams"
)


def _neighbor_barrier(me):
    bar = pltpu.get_barrier_semaphore()
    for k in range(1, N_DEV):
        pl.semaphore_signal(
            bar,
            inc=1,
            device_id=((me + k) % N_DEV,),
            device_id_type=pl.DeviceIdType.MESH,
        )
    pl.semaphore_wait(bar, N_DEV - 1)


def _qproj_allgather(x, Wq):

    def body(x_ref, wq_ref, out_ref, send_sems, recv_sems):
        me = lax.axis_index("i")
        _neighbor_barrier(me)
        q = jnp.dot(x_ref[0], wq_ref[...], preferred_element_type=jnp.float32)
        out_ref[me] = q
        rdmas = []
        for k in range(1, N_DEV):
            r = pltpu.make_async_remote_copy(
                src_ref=out_ref.at[me],
                dst_ref=out_ref.at[me],
                send_sem=send_sems.at[k - 1],
                recv_sem=recv_sems.at[k - 1],
                device_id=((me + k) % N_DEV,),
                device_id_type=pl.DeviceIdType.MESH,
            )
            r.start()
            rdmas.append(r)
        for r in rdmas:
            r.wait_send()
        for r in rdmas:
            r.wait_recv()

    return pl.pallas_call(
        body,
        out_shape=jax.ShapeDtypeStruct((N_DEV, SQ, DM), jnp.float32),
        in_specs=[
            pl.BlockSpec(memory_space=pltpu.VMEM),
            pl.BlockSpec(memory_space=pltpu.VMEM),
        ],
        out_specs=pl.BlockSpec(memory_space=pltpu.VMEM),
        scratch_shapes=[
            pltpu.SemaphoreType.DMA((N_DEV - 1,)),
            pltpu.SemaphoreType.DMA((N_DEV - 1,)),
        ],
        compiler_params=_CompilerParams(collective_id=0),
    )(x, Wq)


def _attn_partials(Qg, K_ext, V_ext):

    NSEL = SKV_L // 4

    def body(q_ref, k_ref, v_ref, acc_ref, ml_ref):
        qb = pl.program_id(1)
        for h in range(HL):
            q = q_ref[0][:, h * DH:(h + 1) * DH]
            k = k_ref[:, 0, :, h, :].reshape(NSEL, DH)
            v = v_ref[:, 0, :, h, :].reshape(NSEL, DH)
            s = (
                lax.dot_general(
                    q, k, (((1,), (1,)), ((), ())),
                    preferred_element_type=jnp.float32,
                )
                * SCALE
            )
            m_t = jnp.max(s, axis=1)
            w = jnp.exp(s - m_t[:, None])
            l_t = jnp.sum(w, axis=1)
            acc_ref[0, h] = lax.dot_general(
                w, v, (((1,), (0,)), ((), ())),
                preferred_element_type=jnp.float32,
            )
            ml_ref[0, 0, h, 0] = m_t
            ml_ref[0, 0, h, 1] = l_t

    return pl.pallas_call(
        body,
        grid=(N_DEV, 4),
        in_specs=[
            pl.BlockSpec((1, 64, DM), lambda j, qb: (j, qb, 0)),
            pl.BlockSpec((16, 1, 64, HL, DH), lambda j, qb: (0, qb, 0, j, 0)),
            pl.BlockSpec((16, 1, 64, HL, DH), lambda j, qb: (0, qb, 0, j, 0)),
        ],
        out_specs=[
            pl.BlockSpec((1, HL, 64, DH), lambda j, qb: (j, 0, qb, 0)),
            pl.BlockSpec((1, 1, HL, 2, 64), lambda j, qb: (j, qb, 0, 0, 0)),
        ],
        out_shape=[
            jax.ShapeDtypeStruct((N_DEV, HL, SQ, DH), jnp.float32),
            jax.ShapeDtypeStruct((N_DEV, 4, HL, 2, 64), jnp.float32),
        ],
    )(Qg, K_ext, V_ext)


def _combine_wo_allreduce(acc, ml, Wo):

    def body(
        acc_ref, ml_ref, wo_ref, out_ref,
        acc_rx, ml_rx, out_rx,
        a_send, a_recv, m_send, m_recv, o_send, o_recv,
    ):
        me = lax.axis_index("i")
        _neighbor_barrier(me)

        acc_rx[me] = acc_ref[me]
        ml_rx[me] = ml_ref[me]
        rdmas = []
        for k in range(1, N_DEV):
            p = (me + k) % N_DEV
            r1 = pltpu.make_async_remote_copy(
                src_ref=acc_ref.at[p],
                dst_ref=acc_rx.at[me],
                send_sem=a_send.at[k - 1],
                recv_sem=a_recv.at[k - 1],
                device_id=(p,),
                device_id_type=pl.DeviceIdType.MESH,
            )
            r2 = pltpu.make_async_remote_copy(
                src_ref=ml_ref.at[p],
                dst_ref=ml_rx.at[me],
                send_sem=m_send.at[k - 1],
                recv_sem=m_recv.at[k - 1],
                device_id=(p,),
                device_id_type=pl.DeviceIdType.MESH,
            )
            r1.start()
            r2.start()
            rdmas += [r1, r2]
        for r in rdmas:
            r.wait_send()
        for r in rdmas:
            r.wait_recv()

        ctx_parts = []
        for qb in range(4):
            m_all = ml_rx[:, qb, :, 0, :]
            l_all = ml_rx[:, qb, :, 1, :]
            big_m = jnp.max(m_all, axis=0)
            sc = jnp.exp(m_all - big_m[None])
            l = jnp.sum(l_all * sc, axis=0)
            a = acc_rx[:, :, qb * 64:(qb + 1) * 64, :]
            ctx_parts.append(
                jnp.sum(a * sc[..., None], axis=0) / l[..., None]
            )
        ctx = jnp.concatenate(ctx_parts, axis=1)

        o = jnp.zeros((SQ, DM), jnp.float32)
        for h in range(HL):
            o = o + jnp.dot(
                ctx[h],
                wo_ref[h * DH:(h + 1) * DH, :],
                preferred_element_type=jnp.float32,
            )

        out_rx[me] = o
        rdmas2 = []
        for k in range(1, N_DEV):
            p = (me + k) % N_DEV
            r = pltpu.make_async_remote_copy(
                src_ref=out_rx.at[me],
                dst_ref=out_rx.at[me],
                send_sem=o_send.at[k - 1],
                recv_sem=o_recv.at[k - 1],
                device_id=(p,),
                device_id_type=pl.DeviceIdType.MESH,
            )
            r.start()
            rdmas2.append(r)
        for r in rdmas2:
            r.wait_send()
        for r in rdmas2:
            r.wait_recv()
        out_ref[...] = jnp.sum(out_rx[...], axis=0)

    return pl.pallas_call(
        body,
        out_shape=jax.ShapeDtypeStruct((SQ, DM), jnp.float32),
        in_specs=[
            pl.BlockSpec(memory_space=pltpu.VMEM),
            pl.BlockSpec(memory_space=pltpu.VMEM),
            pl.BlockSpec(memory_space=pltpu.VMEM),
        ],
        out_specs=pl.BlockSpec(memory_space=pltpu.VMEM),
        scratch_shapes=[
            pltpu.VMEM((N_DEV, HL, SQ, DH), jnp.float32),
            pltpu.VMEM((N_DEV, 4, HL, 2, 64), jnp.float32),
            pltpu.VMEM((N_DEV, SQ, DM), jnp.float32),
            pltpu.SemaphoreType.DMA((N_DEV - 1,)),
            pltpu.SemaphoreType.DMA((N_DEV - 1,)),
            pltpu.SemaphoreType.DMA((N_DEV - 1,)),
            pltpu.SemaphoreType.DMA((N_DEV - 1,)),
            pltpu.SemaphoreType.DMA((N_DEV - 1,)),
            pltpu.SemaphoreType.DMA((N_DEV - 1,)),
        ],
        compiler_params=_CompilerParams(collective_id=1),
    )(acc, ml, Wo)


def kernel(x, Wq, K_ext, V_ext, Wo):
    Qg = _qproj_allgather(x, Wq)
    Kr = K_ext.reshape(SKV_L // 256, 4, 64, 4 * HL, DH)
    Vr = V_ext.reshape(SKV_L // 256, 4, 64, 4 * HL, DH)
    acc, ml = _attn_partials(Qg, Kr, Vr)
    out = _combine_wo_allreduce(acc, ml, Wo)
    return out.reshape(1, SQ, DM)
